# Initial kernel scaffold; baseline (speedup 1.0000x reference)
import jax
import jax.numpy as jnp
from jax.experimental import pallas as pl


def kernel(x, emb_w, lin_w, bias):
    raise NotImplementedError("write your pallas kernel here")


# speedup vs baseline: 2.1196x; 2.1196x over previous
import functools

import jax
import jax.numpy as jnp
from jax import lax
from jax.experimental import pallas as pl
from jax.experimental.pallas import tpu as pltpu
from jax.experimental.pallas import tpu_sc as plsc

B = 16384
F = 26
D = 32
NW = 32
SPW = B // NW
CH = 64
NCHUNK = SPW // CH
G = 128
NG = CH * F // G
ROWS = CH * F


def _fm_body(x_hbm, emb_hbm, lin_hbm, bias_hbm, out_hbm,
             idx_v, rows_v, lin_v, out_v, bias_v, sem):
    cid = lax.axis_index("c")
    sid = lax.axis_index("s")
    wid = cid * 16 + sid

    pltpu.sync_copy(bias_hbm, bias_v.at[pl.ds(0, 1)])
    bias = bias_v[...][0]

    iota16 = lax.iota(jnp.int32, 16)
    lin_mask = iota16 < (F - 16)

    def chunk_body(c, carry):
        gc = wid * NCHUNK + c
        pltpu.sync_copy(x_hbm.at[pl.ds(gc * ROWS, ROWS)], idx_v)

        def fire(j, carry2):
            pltpu.make_async_copy(
                emb_hbm.at[idx_v.at[pl.ds(j * G, G)]],
                rows_v.at[pl.ds(j * G, G)], sem
            ).start()
            pltpu.make_async_copy(
                lin_hbm.at[idx_v.at[pl.ds(j * G, G)]],
                lin_v.at[pl.ds(j * G, G)], sem
            ).start()
            return carry2

        lax.fori_loop(0, NG, fire, None)

        def drain(j, carry2):
            pltpu.make_async_copy(
                emb_hbm.at[idx_v.at[pl.ds(j * G, G)]],
                rows_v.at[pl.ds(j * G, G)], sem
            ).wait()
            pltpu.make_async_copy(
                lin_hbm.at[idx_v.at[pl.ds(j * G, G)]],
                lin_v.at[pl.ds(j * G, G)], sem
            ).wait()
            return carry2

        lax.fori_loop(0, NG, drain, None)

        def group_body(g, carry2):
            def sample_body(s16, vec):
                s = g * 16 + s16
                rb = s * F
                acc0 = jnp.zeros((16,), jnp.float32)
                acc1 = jnp.zeros((16,), jnp.float32)
                sq0 = jnp.zeros((16,), jnp.float32)
                sq1 = jnp.zeros((16,), jnp.float32)
                for f in range(F):
                    v0 = rows_v[rb + f, pl.ds(0, 16)]
                    v1 = rows_v[rb + f, pl.ds(16, 16)]
                    acc0 = acc0 + v0
                    acc1 = acc1 + v1
                    sq0 = sq0 + v0 * v0
                    sq1 = sq1 + v1 * v1
                l0 = lin_v[pl.ds(rb, 16)]
                l1 = lin_v[pl.ds(rb + 16, 16)]
                lin = jnp.sum(l0 + jnp.where(lin_mask, l1, 0.0))
                ps = jnp.sum(acc0 * acc0 + acc1 * acc1)
                sp = jnp.sum(sq0 + sq1)
                val = 0.5 * (ps - sp) + lin + bias
                return jnp.where(iota16 == s16, val, vec)

            vec = lax.fori_loop(0, 16, sample_body, jnp.zeros((16,), jnp.float32))
            out_v[pl.ds(c * CH + g * 16, 16)] = vec
            return carry2

        lax.fori_loop(0, CH // 16, group_body, None)
        return carry

    lax.fori_loop(0, NCHUNK, chunk_body, None)
    pltpu.sync_copy(out_v, out_hbm.at[pl.ds(wid * SPW, SPW)])


_fm = functools.partial(
    pl.kernel,
    mesh=plsc.VectorSubcoreMesh(core_axis_name="c", subcore_axis_name="s"),
    out_type=jax.ShapeDtypeStruct((B,), jnp.float32),
    scratch_types=[
        pltpu.VMEM((ROWS,), jnp.int32),
        pltpu.VMEM((ROWS, D), jnp.float32),
        pltpu.VMEM((ROWS + 16,), jnp.float32),
        pltpu.VMEM((SPW,), jnp.float32),
        pltpu.VMEM((16,), jnp.float32),
        pltpu.SemaphoreType.DMA,
    ],
    compiler_params=pltpu.CompilerParams(
        needs_layout_passes=False, use_tc_tiling_on_sc=False
    ),
)(_fm_body)


def kernel(x, emb_w, lin_w, bias):
    x2 = x.astype(jnp.int32).reshape(B * F)
    lin_flat = lin_w.reshape(-1)
    out = _fm(x2, emb_w, lin_flat, bias)
    return out.reshape(B, 1)
